# Initial kernel scaffold; baseline (speedup 1.0000x reference)
import jax
import jax.numpy as jnp
from jax.experimental import pallas as pl


def kernel(batch, seg_len, concept1, concept2, W_enc, b_enc, W_t1, b_t1, W_t2, b_t2, topic_emb, W_c, b_c):
    raise NotImplementedError("write your pallas kernel here")


# speedup vs baseline: 3.7892x; 3.7892x over previous
import functools

import jax
import jax.numpy as jnp
from jax.experimental import pallas as pl
from jax.experimental.pallas import tpu as pltpu


def _pool_body(seg_ref, x_ref, out_ref):
    b = pl.program_id(0)
    s = pl.program_id(1)
    l = seg_ref[b, s]
    x = x_ref[0, 0]
    rows = jax.lax.broadcasted_iota(jnp.int32, (x.shape[0], 1), 0)
    xm = jnp.where(rows < l, x, 0.0)
    part = jnp.sum(xm, axis=0, keepdims=True)[None]

    @pl.when(s == 0)
    def _init():
        out_ref[...] = part

    @pl.when(s != 0)
    def _acc():
        out_ref[...] += part


def _mlp_body(segf_ref, pooled_ref, c1_ref, c2_ref, Wenc_ref, benc_ref,
              Wt1_ref, bt1_ref, Wt2_ref, bt2_ref, temb_ref, Wc_ref, bc_ref,
              q_ref, tp_ref):
    TE = temb_ref.shape[1]
    count = jnp.sum(segf_ref[...], axis=1, keepdims=True)
    pooled = pooled_ref[...] / count
    vf = jax.nn.relu(
        jnp.dot(pooled, Wenc_ref[...], preferred_element_type=jnp.float32)
        + benc_ref[...])
    cat = jnp.concatenate([c1_ref[...], c2_ref[...], vf], axis=1)
    h = jax.nn.relu(
        jnp.dot(cat, Wt1_ref[...], preferred_element_type=jnp.float32)
        + bt1_ref[...])
    logits = jnp.dot(h, Wt2_ref[...], preferred_element_type=jnp.float32) + bt2_ref[...]
    m = jnp.max(logits, axis=1, keepdims=True)
    e = jnp.exp(logits - m)
    tp_ref[...] = (e / jnp.sum(e, axis=1, keepdims=True))[:, None, :]
    E_T = jax.lax.dot_general(Wc_ref[0:TE, :], temb_ref[...],
                              dimension_numbers=(((0,), (1,)), ((), ())),
                              preferred_element_type=jnp.float32)
    catv = jnp.concatenate([vf, c1_ref[...], c2_ref[...]], axis=1)
    V = jnp.dot(catv, Wc_ref[TE:, :], preferred_element_type=jnp.float32) + bc_ref[...]
    q_ref[...] = jax.nn.relu(V[:, :, None] + E_T[None, :, :])


def _score_body(seg_ref, x_ref, q_ref, tp_ref, out_ref, *, tn):
    b = pl.program_id(0)
    s = pl.program_id(1)
    l = seg_ref[b, s]
    x = x_ref[0, 0]
    q = q_ref[0]
    L = jnp.dot(x, q, preferred_element_type=jnp.float32)
    sc = jax.nn.sigmoid(L) * tp_ref[0]
    sc = jax.nn.relu(sc - 0.01)
    tot = jnp.sum(sc, axis=1, keepdims=True) * (1.0 / tn)
    rows = jax.lax.broadcasted_iota(jnp.int32, tot.shape, 0)
    tot = jnp.where(rows < l, tot, 0.0)
    out_ref[...] = tot.reshape(1, 1, 1, -1)


def kernel(batch, seg_len, concept1, concept2, W_enc, b_enc, W_t1, b_t1,
           W_t2, b_t2, topic_emb, W_c, b_c):
    B, S, F, D = batch.shape
    TN, TE = topic_emb.shape
    SH = W_enc.shape[1]
    CD = concept1.shape[1]

    seg_len = seg_len.astype(jnp.int32)

    sums = pl.pallas_call(
        _pool_body,
        grid_spec=pltpu.PrefetchScalarGridSpec(
            num_scalar_prefetch=1,
            grid=(B, S),
            in_specs=[pl.BlockSpec((1, 1, F, D), lambda b, s, seg: (b, s, 0, 0))],
            out_specs=pl.BlockSpec((1, 1, D), lambda b, s, seg: (b, 0, 0)),
        ),
        out_shape=jax.ShapeDtypeStruct((B, 1, D), jnp.float32),
        compiler_params=pltpu.CompilerParams(
            dimension_semantics=("arbitrary", "arbitrary")),
    )(seg_len, batch)

    q, tp = pl.pallas_call(
        _mlp_body,
        out_shape=(jax.ShapeDtypeStruct((B, D, TN), jnp.float32),
                   jax.ShapeDtypeStruct((B, 1, TN), jnp.float32)),
    )(seg_len.astype(jnp.float32), sums.reshape(B, D), concept1, concept2,
      W_enc, b_enc.reshape(1, SH), W_t1, b_t1.reshape(1, -1),
      W_t2, b_t2.reshape(1, TN), topic_emb, W_c, b_c.reshape(1, D))

    overall = pl.pallas_call(
        functools.partial(_score_body, tn=float(TN)),
        grid_spec=pltpu.PrefetchScalarGridSpec(
            num_scalar_prefetch=1,
            grid=(B, S),
            in_specs=[
                pl.BlockSpec((1, 1, F, D), lambda b, s, seg: (b, s, 0, 0)),
                pl.BlockSpec((1, D, TN), lambda b, s, seg: (b, 0, 0)),
                pl.BlockSpec((1, 1, TN), lambda b, s, seg: (b, 0, 0)),
            ],
            out_specs=pl.BlockSpec((1, 1, 1, F), lambda b, s, seg: (b, s, 0, 0)),
        ),
        out_shape=jax.ShapeDtypeStruct((B, S, 1, F), jnp.float32),
        compiler_params=pltpu.CompilerParams(
            dimension_semantics=("arbitrary", "arbitrary")),
    )(seg_len, batch, q, tp)

    overall = overall.reshape(B, S, F)
    return (overall, overall)
